# baseline (speedup 1.0000x reference)
import functools

import jax
import jax.numpy as jnp
from jax import lax
from jax.experimental import pallas as pl
from jax.experimental.pallas import tpu as pltpu
from jax.experimental.pallas import tpu_sc as plsc

D = 64
BATCH = 4096
HIST = 200
B = BATCH * HIST

NC = 2
NS = 16
NW = NC * NS
PW = B // NW
CHUNK = 128
NCHUNK = PW // CHUNK
LANES = 16


VOCAB = 1000000
NBLK = VOCAB // 128
BLK_PW = NBLK // NW


def _repack_body(tt_hbm, tail_hbm, t128_hbm, sin_v, sout_v, rsem, wsem):
    wid = lax.axis_index("s") * NC + lax.axis_index("c")
    d_idx = [lax.iota(jnp.int32, 16) + 16 * k for k in range(D // LANES)]

    def start(blk):
        return pl.multiple_of(blk * 128, 128)

    def rd_issue(blk, b):
        pltpu.async_copy(tt_hbm.at[:, pl.ds(start(blk), 128)], sin_v.at[b],
                         rsem.at[b])

    def rd_wait(b):
        pltpu.make_async_copy(tt_hbm.at[:, pl.ds(0, 128)], sin_v.at[b],
                              rsem.at[b]).wait()

    def wr_issue(blk, b):
        pltpu.async_copy(sout_v.at[b], t128_hbm.at[pl.ds(start(blk), 128)],
                         wsem.at[b])

    def wr_wait(b):
        pltpu.make_async_copy(sout_v.at[b], t128_hbm.at[pl.ds(0, 128)],
                              wsem.at[b]).wait()

    def transpose(b):
        def rbody(r, _):
            col = jnp.full((16,), r, jnp.int32)
            for k in range(D // LANES):
                v = plsc.load_gather(sin_v.at[b], [d_idx[k], col])
                sout_v[b, r, pl.ds(16 * k, LANES)] = v
            return 0
        lax.fori_loop(0, 128, rbody, 0)

    def blk_of(t, w):
        return w + NW * t

    rd_issue(blk_of(0, wid), 0)
    rd_issue(blk_of(1, wid), 1)

    def body(t, b, wait_w, more):
        rd_wait(b)
        if wait_w:
            wr_wait(b)
        transpose(b)
        wr_issue(blk_of(t, wid), b)
        if more:
            rd_issue(blk_of(t + 2, wid), b)

    body(0, 0, False, True)
    body(1, 1, False, True)

    def outer(u, _):
        body(2 * u, 0, True, True)
        body(2 * u + 1, 1, True, True)
        return 0

    lax.fori_loop(1, BLK_PW // 2 - 1, outer, 0)
    body(BLK_PW - 2, 0, True, False)
    body(BLK_PW - 1, 1, True, False)

    @pl.when(wid < NBLK - NW * BLK_PW)
    def _():
        rd_issue(NW * BLK_PW + wid, 0)
        rd_wait(0)
        wr_wait(0)
        transpose(0)
        wr_issue(NW * BLK_PW + wid, 0)

    @pl.when(wid == NW - 1)
    def _():
        wr_wait(1)
        pltpu.sync_copy(tail_hbm, sin_v.at[1])
        pltpu.sync_copy(sin_v.at[1], t128_hbm.at[pl.ds(VOCAB - 64, 64)])

    wr_wait(0)

    @pl.when(wid != NW - 1)
    def _():
        wr_wait(1)


def _sc_body(x_hbm, table_hbm, out_hbm, pidx_v, rows_v, sout_v,
             isem, gsem, osem):
    wid = lax.axis_index("s") * NC + lax.axis_index("c")
    base = wid * PW

    def idx_issue(c, b):
        pltpu.async_copy(
            x_hbm.at[pl.ds(base + c * CHUNK, CHUNK)],
            pidx_v.at[b].at[pl.ds(0, CHUNK)], isem.at[b])

    def idx_wait(b):
        pltpu.make_async_copy(
            x_hbm.at[pl.ds(base, CHUNK)],
            pidx_v.at[b].at[pl.ds(0, CHUNK)], isem.at[b]).wait()

    def gather_issue(b):
        pltpu.async_copy(
            table_hbm.at[pidx_v.at[b].at[pl.ds(0, CHUNK)]], rows_v.at[b],
            gsem.at[b])

    def gather_wait(b):
        pltpu.make_async_copy(
            table_hbm.at[pidx_v.at[b].at[pl.ds(0, CHUNK)]], rows_v.at[b],
            gsem.at[b]).wait()

    def wo_issue(c, b):
        pltpu.async_copy(
            sout_v.at[b], out_hbm.at[pl.ds(base + c * CHUNK, CHUNK)],
            osem.at[b])

    def wo_wait(b):
        pltpu.make_async_copy(
            sout_v.at[b], out_hbm.at[pl.ds(base, CHUNK)], osem.at[b]).wait()

    def compute(b):
        def rbody(i, _):
            for j in range(D // LANES):
                v = rows_v[b, i, pl.ds(j * LANES, LANES)]
                sout_v[b, i, pl.ds(j * LANES, LANES)] = jnp.sign(v)
            return 0
        lax.fori_loop(0, CHUNK, rbody, 0)

    def chunk_body(c, b, prep_next, wait_wo, stage_next):
        if prep_next:
            b1 = (c + 1) % 2
            idx_wait(b1)
            gather_issue(b1)
        gather_wait(b)
        if wait_wo:
            wo_wait(b)
        compute(b)
        wo_issue(c, b)
        if stage_next:
            idx_issue(c + 2, b)

    idx_issue(0, 0)
    idx_wait(0)
    gather_issue(0)
    idx_issue(1, 1)

    chunk_body(0, 0, True, False, True)
    chunk_body(1, 1, True, False, True)

    def outer(t, _):
        c0 = t * 2
        chunk_body(c0, 0, True, True, True)
        chunk_body(c0 + 1, 1, True, True, True)
        return 0

    lax.fori_loop(1, NCHUNK // 2 - 1, outer, 0)

    c0 = NCHUNK - 2
    chunk_body(c0, 0, True, True, False)
    chunk_body(c0 + 1, 1, False, True, False)

    wo_wait(0)
    wo_wait(1)


@functools.partial(jax.jit, static_argnames=())
def kernel(x, table):
    x_flat = x.reshape(-1)
    mesh = plsc.VectorSubcoreMesh(core_axis_name="c", subcore_axis_name="s")
    table2 = pl.kernel(
        _repack_body,
        mesh=mesh,
        compiler_params=pltpu.CompilerParams(needs_layout_passes=False),
        out_type=jax.ShapeDtypeStruct((VOCAB, 2 * D), jnp.float32),
        scratch_types=[
            pltpu.VMEM((2, D, 128), jnp.float32),
            pltpu.VMEM((2, 128, 128), jnp.float32),
            pltpu.SemaphoreType.DMA((2,)),
            pltpu.SemaphoreType.DMA((2,)),
        ],
    )(table.T, jnp.pad(table[VOCAB - 64:, :], ((0, 0), (0, D))))
    out = pl.kernel(
        _sc_body,
        mesh=mesh,
        out_type=jax.ShapeDtypeStruct((B, D), jnp.float32),
        scratch_types=[
            pltpu.VMEM((2, 2 * CHUNK), jnp.int32),
            pltpu.VMEM((2, CHUNK, 2 * D), jnp.float32),
            pltpu.VMEM((2, CHUNK, D), jnp.float32),
            pltpu.SemaphoreType.DMA((2,)),
            pltpu.SemaphoreType.DMA((2,)),
            pltpu.SemaphoreType.DMA((2,)),
        ],
    )(x_flat, table2)
    return out.reshape(BATCH, HIST, D)


# speedup vs baseline: 1.9261x; 1.9261x over previous
import functools

import jax
import jax.numpy as jnp
from jax import lax
from jax.experimental import pallas as pl
from jax.experimental.pallas import tpu as pltpu
from jax.experimental.pallas import tpu_sc as plsc

D = 64
BATCH = 4096
HIST = 200
B = BATCH * HIST

NC = 2
NS = 16
NW = NC * NS
PW = B // NW
CHUNK = 128
NCHUNK = PW // CHUNK
LANES = 16


def _sc_body(x_hbm, table_hbm, out_hbm, pidx_v, rows_v, sout_v,
             isem, gsem, osem):
    wid = lax.axis_index("s") * NC + lax.axis_index("c")
    base = wid * PW

    def idx_issue(c, b):
        pltpu.async_copy(
            x_hbm.at[pl.ds(base + c * CHUNK, CHUNK)],
            pidx_v.at[b].at[pl.ds(0, CHUNK)], isem.at[b])

    def idx_wait(b):
        pltpu.make_async_copy(
            x_hbm.at[pl.ds(base, CHUNK)],
            pidx_v.at[b].at[pl.ds(0, CHUNK)], isem.at[b]).wait()

    def gather_issue(b):
        pltpu.async_copy(
            table_hbm.at[pidx_v.at[b].at[pl.ds(0, CHUNK)]], rows_v.at[b],
            gsem.at[b])

    def gather_wait(b):
        pltpu.make_async_copy(
            table_hbm.at[pidx_v.at[b].at[pl.ds(0, CHUNK)]], rows_v.at[b],
            gsem.at[b]).wait()

    def wo_issue(c, b):
        pltpu.async_copy(
            sout_v.at[b], out_hbm.at[pl.ds(base + c * CHUNK, CHUNK)],
            osem.at[b])

    def wo_wait(b):
        pltpu.make_async_copy(
            sout_v.at[b], out_hbm.at[pl.ds(base, CHUNK)], osem.at[b]).wait()

    def compute(b):
        def rbody(i, _):
            for j in range(D // LANES):
                v = rows_v[b, i, pl.ds(j * LANES, LANES)]
                sout_v[b, i, pl.ds(j * LANES, LANES)] = jnp.sign(v)
            return 0
        lax.fori_loop(0, CHUNK, rbody, 0)

    def chunk_body(c, b, prep_next, wait_wo, stage_next):
        if prep_next:
            b1 = (c + 1) % 2
            idx_wait(b1)
            gather_issue(b1)
        gather_wait(b)
        if wait_wo:
            wo_wait(b)
        compute(b)
        wo_issue(c, b)
        if stage_next:
            idx_issue(c + 2, b)

    idx_issue(0, 0)
    idx_wait(0)
    gather_issue(0)
    idx_issue(1, 1)

    chunk_body(0, 0, True, False, True)
    chunk_body(1, 1, True, False, True)

    def outer(t, _):
        c0 = t * 2
        chunk_body(c0, 0, True, True, True)
        chunk_body(c0 + 1, 1, True, True, True)
        return 0

    lax.fori_loop(1, NCHUNK // 2 - 1, outer, 0)

    c0 = NCHUNK - 2
    chunk_body(c0, 0, True, True, False)
    chunk_body(c0 + 1, 1, False, True, False)

    wo_wait(0)
    wo_wait(1)


@functools.partial(jax.jit, static_argnames=())
def kernel(x, table):
    x_flat = x.reshape(-1)
    mesh = plsc.VectorSubcoreMesh(core_axis_name="c", subcore_axis_name="s")
    table2 = jnp.pad(table, ((0, 0), (0, D)))
    out = pl.kernel(
        _sc_body,
        mesh=mesh,
        out_type=jax.ShapeDtypeStruct((B, D), jnp.float32),
        scratch_types=[
            pltpu.VMEM((2, 2 * CHUNK), jnp.int32),
            pltpu.VMEM((2, CHUNK, 2 * D), jnp.float32),
            pltpu.VMEM((2, CHUNK, D), jnp.float32),
            pltpu.SemaphoreType.DMA((2,)),
            pltpu.SemaphoreType.DMA((2,)),
            pltpu.SemaphoreType.DMA((2,)),
        ],
    )(x_flat, table2)
    return out.reshape(BATCH, HIST, D)
